# Initial kernel scaffold; baseline (speedup 1.0000x reference)
import jax
import jax.numpy as jnp
from jax.experimental import pallas as pl


def kernel(logits, topk_log_probs, growing_beam, step):
    raise NotImplementedError("write your pallas kernel here")


# speedup vs baseline: 1.1227x; 1.1227x over previous
import functools

import jax
import jax.numpy as jnp
from jax.experimental import pallas as pl
from jax.experimental.pallas import tpu as pltpu

_BEAM = 8
_VOCAB = 100000
_W = 512
_NFULL = _VOCAB // _W
_TAIL = _VOCAB - _NFULL * _W
_END = 2
_MINLEN = 10
_NEG = -1e30
_BIGI = 1 << 30


def _body(step_ref, x_ref, tlp_ref, gb_ref,
          sc_ref, lp_ref, tok_ref, row_ref, nb_ref, fin_ref,
          lps_ref, gs_ref):
    step = step_ref[0, 0]
    masking = step < _MINLEN
    pid = pl.program_id(0)

    lane = jax.lax.broadcasted_iota(jnp.int32, (_BEAM, _W), 1)
    row8 = jax.lax.broadcasted_iota(jnp.int32, (_BEAM, _W), 0)

    x0 = x_ref[:, pl.ds(0, _W)]
    x0 = jnp.where((lane == _END) & masking, jnp.float32(-1e20), x0)
    v1, b1 = x0, jnp.zeros((_BEAM, _W), jnp.int32)
    v2 = jnp.full((_BEAM, _W), _NEG, jnp.float32)
    b2 = jnp.zeros((_BEAM, _W), jnp.int32)

    def upd(base, x, carry):
        v1, b1, v2, b2 = carry
        gt1 = x > v1
        gt2 = x > v2
        nv1 = jnp.maximum(x, v1)
        nb1 = jnp.where(gt1, base, b1)
        nv2 = jnp.where(gt1, v1, jnp.where(gt2, x, v2))
        nb2 = jnp.where(gt1, b1, jnp.where(gt2, base, b2))
        return nv1, nb1, nv2, nb2

    def stepA(c, carry):
        base = pl.multiple_of(c * _W, _W)
        x = x_ref[:, pl.ds(base, _W)]
        return upd(base, x, carry)

    v1, b1, v2, b2 = jax.lax.fori_loop(1, _NFULL, stepA, (v1, b1, v2, b2))

    xt = x_ref[:, pl.ds(_NFULL * _W, _TAIL)]
    xt = jnp.concatenate(
        [xt, jnp.full((_BEAM, _W - _TAIL), _NEG, jnp.float32)], axis=1)
    v1, b1, v2, b2 = upd(_NFULL * _W, xt, (v1, b1, v2, b2))

    m_row = jnp.max(v1, axis=1, keepdims=True)
    m_row = jnp.maximum(m_row, x_ref[:, pl.ds(_END, 1)])

    def stepB(c, acc):
        x = x_ref[:, pl.ds(pl.multiple_of(c * _W, _W), _W)]
        return acc + jnp.exp(x - m_row)

    acc = jax.lax.fori_loop(0, _NFULL, stepB, jnp.zeros((_BEAM, _W), jnp.float32))
    acc = acc + jnp.exp(xt - m_row)
    logS = jnp.log(jnp.sum(acc, axis=1, keepdims=True))

    tlp = tlp_ref[...]
    lp1 = ((v1 - m_row) - logS) + tlp
    lp2 = ((v2 - m_row) - logS) + tlp
    g1 = row8 * _VOCAB + b1 + lane
    g2 = row8 * _VOCAB + b2 + lane
    lps_ref[...] = jnp.concatenate([lp1, lp2], axis=0)
    gs_ref[...] = jnp.concatenate([g1, g2], axis=0)

    sf = (jnp.float32(6.0) + step.astype(jnp.float32)) / jnp.float32(6.0)
    pen = jnp.exp(jnp.float32(0.95) * jnp.log(sf))

    lane16 = jax.lax.broadcasted_iota(jnp.int32, (2 * _BEAM, _W), 1)
    row16 = jax.lax.broadcasted_iota(jnp.int32, (2 * _BEAM, _W), 0)
    gbrow8 = jax.lax.broadcasted_iota(jnp.int32, (_BEAM, 16), 0)

    winners_b = []
    picked_g = []

    for i in range(_BEAM):
        lp_all = lps_ref[...]
        g_all = gs_ref[...]
        w = jnp.max(lp_all)
        gw = jnp.min(jnp.where(lp_all == w, g_all, _BIGI))
        r = gw // _VOCAB
        pos = gw - r * _VOCAB
        l = pos % _W
        bid = r * _W + l
        picked_g.append(gw)

        lps_ref[...] = jnp.where(g_all == gw, jnp.float32(_NEG), lp_all)

        exhausted = jnp.bool_(False)
        for pb in winners_b:
            exhausted = jnp.logical_or(exhausted, pb == bid)
        winners_b.append(bid)

        if i > 0:
            @pl.when(exhausted)
            def _rescan(r=r, l=l, picked=tuple(picked_g)):
                def excl(base, x):
                    g = row8 * _VOCAB + base + lane
                    m = masking & (base + lane == _END)
                    for pg in picked:
                        m = m | (g == pg)
                    return jnp.where(m, jnp.float32(_NEG), x)

                def stepR(c, carry):
                    vm, jm = carry
                    base = pl.multiple_of(c * _W, _W)
                    x = excl(base, x_ref[:, pl.ds(base, _W)])
                    upd_ = x > vm
                    return jnp.maximum(x, vm), jnp.where(upd_, base, jm)

                vm = jnp.full((_BEAM, _W), _NEG, jnp.float32)
                jm = jnp.zeros((_BEAM, _W), jnp.int32)
                vm, jm = jax.lax.fori_loop(0, _NFULL, stepR, (vm, jm))
                xte = excl(_NFULL * _W, xt)
                updt = xte > vm
                vm = jnp.maximum(xte, vm)
                jm = jnp.where(updt, _NFULL * _W, jm)

                lp_new = ((vm - m_row) - logS) + tlp
                g_new = row8 * _VOCAB + jm + lane
                sel = (row16 == r) & (lane16 == l)
                lps_ref[...] = jnp.where(
                    sel, jnp.tile(lp_new, (2, 1)), lps_ref[...])
                gs_ref[...] = jnp.where(
                    sel, jnp.tile(g_new, (2, 1)), gs_ref[...])

        tok = pos
        sc_ref[pl.ds(i, 1), :] = jnp.reshape(w / pen, (1, 1))
        lp_ref[pl.ds(i, 1), :] = jnp.reshape(w, (1, 1))
        tok_ref[pl.ds(i, 1), :] = jnp.reshape(tok, (1, 1))
        row_ref[pl.ds(i, 1), :] = jnp.reshape(r + pid * _BEAM, (1, 1))
        fin_ref[pl.ds(i, 1), :] = jnp.reshape(
            (tok == _END).astype(jnp.int32), (1, 1))
        gbrow = jnp.sum(
            jnp.where(gbrow8 == r, gb_ref[...], 0), axis=0, keepdims=True)
        nb_ref[pl.ds(i, 1), pl.ds(0, 16)] = gbrow
        nb_ref[pl.ds(i, 1), pl.ds(16, 1)] = jnp.reshape(tok, (1, 1))


def kernel(logits, topk_log_probs, growing_beam, step):
    nrows = logits.shape[0]
    nb = nrows // _BEAM
    cur_len = growing_beam.shape[1]
    step2d = jnp.reshape(jnp.asarray(step, jnp.int32), (1, 1))
    tlp2d = jnp.reshape(topk_log_probs, (nrows, 1))

    out_shape = [
        jax.ShapeDtypeStruct((nrows, 1), jnp.float32),
        jax.ShapeDtypeStruct((nrows, 1), jnp.float32),
        jax.ShapeDtypeStruct((nrows, 1), jnp.int32),
        jax.ShapeDtypeStruct((nrows, 1), jnp.int32),
        jax.ShapeDtypeStruct((nrows, cur_len + 1), jnp.int32),
        jax.ShapeDtypeStruct((nrows, 1), jnp.int32),
    ]
    sc, lp, tok, rows, nbm, fin = pl.pallas_call(
        _body,
        grid=(nb,),
        in_specs=[
            pl.BlockSpec(memory_space=pltpu.SMEM),
            pl.BlockSpec((_BEAM, _VOCAB), lambda b: (b, 0)),
            pl.BlockSpec((_BEAM, 1), lambda b: (b, 0)),
            pl.BlockSpec((_BEAM, cur_len), lambda b: (b, 0)),
        ],
        out_specs=[
            pl.BlockSpec((_BEAM, 1), lambda b: (b, 0)),
            pl.BlockSpec((_BEAM, 1), lambda b: (b, 0)),
            pl.BlockSpec((_BEAM, 1), lambda b: (b, 0)),
            pl.BlockSpec((_BEAM, 1), lambda b: (b, 0)),
            pl.BlockSpec((_BEAM, cur_len + 1), lambda b: (b, 0)),
            pl.BlockSpec((_BEAM, 1), lambda b: (b, 0)),
        ],
        scratch_shapes=[
            pltpu.VMEM((2 * _BEAM, _W), jnp.float32),
            pltpu.VMEM((2 * _BEAM, _W), jnp.int32),
        ],
        out_shape=out_shape,
    )(step2d, logits, tlp2d, growing_beam)

    return (sc.reshape(nb, _BEAM), lp.reshape(nb, _BEAM),
            tok.reshape(nb, _BEAM), rows.reshape(-1), nbm,
            (fin.reshape(nb, _BEAM) != 0))


# speedup vs baseline: 2.4949x; 2.2221x over previous
import functools

import jax
import jax.numpy as jnp
from jax.experimental import pallas as pl
from jax.experimental.pallas import tpu as pltpu

_BEAM = 8
_VOCAB = 100000
_W = 512
_NFULL = _VOCAB // _W
_TAIL = _VOCAB - _NFULL * _W
_END = 2
_MINLEN = 10
_NEG = -1e30
_BIGI = 1 << 30


def _body(step_ref, pen_ref, x_ref, tlp_ref, gb_ref,
          sc_ref, lp_ref, tok_ref, row_ref, nb_ref, fin_ref,
          lps_ref, gs_ref, mb_ref):
    step = step_ref[0, 0]
    inv_pen = pen_ref[0, 0]
    masking = step < _MINLEN
    pid = pl.program_id(0)

    lane = jax.lax.broadcasted_iota(jnp.int32, (_BEAM, _W), 1)
    row8 = jax.lax.broadcasted_iota(jnp.int32, (_BEAM, _W), 0)

    x0 = x_ref[:, pl.ds(0, _W)]
    x0 = jnp.where((lane == _END) & masking, jnp.float32(-1e20), x0)
    v1, b1 = x0, jnp.zeros((_BEAM, _W), jnp.int32)
    v2 = jnp.full((_BEAM, _W), _NEG, jnp.float32)
    b2 = jnp.zeros((_BEAM, _W), jnp.int32)

    def upd(base, x, carry):
        v1, b1, v2, b2 = carry
        gt1 = x > v1
        gt2 = x > v2
        nv1 = jnp.maximum(x, v1)
        nb1 = jnp.where(gt1, base, b1)
        nv2 = jnp.where(gt1, v1, jnp.where(gt2, x, v2))
        nb2 = jnp.where(gt1, b1, jnp.where(gt2, base, b2))
        return nv1, nb1, nv2, nb2

    def stepA(c, carry):
        base = pl.multiple_of(c * _W, _W)
        x = x_ref[:, pl.ds(base, _W)]
        return upd(base, x, carry)

    v1, b1, v2, b2 = jax.lax.fori_loop(1, _NFULL, stepA, (v1, b1, v2, b2))

    xt = x_ref[:, pl.ds(_NFULL * _W, _TAIL)]
    xt = jnp.concatenate(
        [xt, jnp.full((_BEAM, _W - _TAIL), _NEG, jnp.float32)], axis=1)
    v1, b1, v2, b2 = upd(_NFULL * _W, xt, (v1, b1, v2, b2))

    m_row = jnp.max(v1, axis=1, keepdims=True)
    m_row = jnp.maximum(m_row, x_ref[:, pl.ds(_END, 1)])
    mb_ref[...] = jnp.broadcast_to(m_row, (_BEAM, _W))
    m_bc = mb_ref[...]

    def stepB(c, acc):
        x = x_ref[:, pl.ds(pl.multiple_of(c * _W, _W), _W)]
        return acc + jnp.exp(x - m_bc)

    acc = jax.lax.fori_loop(0, _NFULL, stepB, jnp.zeros((_BEAM, _W), jnp.float32))
    acc = acc + jnp.exp(xt - m_bc)
    logS = jnp.log(jnp.sum(acc, axis=1, keepdims=True))

    tlp = tlp_ref[...]
    lp1 = ((v1 - m_row) - logS) + tlp
    lp2 = ((v2 - m_row) - logS) + tlp
    g1 = row8 * _VOCAB + b1 + lane
    g2 = row8 * _VOCAB + b2 + lane
    lps_ref[...] = jnp.concatenate([lp1, lp2], axis=0)
    gs_ref[...] = jnp.concatenate([g1, g2], axis=0)

    lane16 = jax.lax.broadcasted_iota(jnp.int32, (2 * _BEAM, _W), 1)
    row16 = jax.lax.broadcasted_iota(jnp.int32, (2 * _BEAM, _W), 0)
    gbrow8 = jax.lax.broadcasted_iota(jnp.int32, (_BEAM, 16), 0)

    winners_b = []
    picked_g = []

    for i in range(_BEAM):
        lp_all = lps_ref[...]
        g_all = gs_ref[...]
        w = jnp.max(lp_all)
        gw = jnp.min(jnp.where(lp_all == w, g_all, _BIGI))
        r = gw // _VOCAB
        pos = gw - r * _VOCAB
        l = pos % _W
        bid = r * _W + l
        picked_g.append(gw)

        lps_ref[...] = jnp.where(g_all == gw, jnp.float32(_NEG), lp_all)

        exhausted = jnp.bool_(False)
        for pb in winners_b:
            exhausted = jnp.logical_or(exhausted, pb == bid)
        winners_b.append(bid)

        if i > 0:
            @pl.when(exhausted)
            def _rescan(r=r, l=l, picked=tuple(picked_g)):
                def excl(base, x):
                    g = row8 * _VOCAB + base + lane
                    m = masking & (base + lane == _END)
                    for pg in picked:
                        m = m | (g == pg)
                    return jnp.where(m, jnp.float32(_NEG), x)

                def stepR(c, carry):
                    vm, jm = carry
                    base = pl.multiple_of(c * _W, _W)
                    x = excl(base, x_ref[:, pl.ds(base, _W)])
                    upd_ = x > vm
                    return jnp.maximum(x, vm), jnp.where(upd_, base, jm)

                vm = jnp.full((_BEAM, _W), _NEG, jnp.float32)
                jm = jnp.zeros((_BEAM, _W), jnp.int32)
                vm, jm = jax.lax.fori_loop(0, _NFULL, stepR, (vm, jm))
                xte = excl(_NFULL * _W, xt)
                updt = xte > vm
                vm = jnp.maximum(xte, vm)
                jm = jnp.where(updt, _NFULL * _W, jm)

                lp_new = ((vm - m_row) - logS) + tlp
                g_new = row8 * _VOCAB + jm + lane
                sel = (row16 == r) & (lane16 == l)
                lps_ref[...] = jnp.where(
                    sel, jnp.tile(lp_new, (2, 1)), lps_ref[...])
                gs_ref[...] = jnp.where(
                    sel, jnp.tile(g_new, (2, 1)), gs_ref[...])

        tok = pos
        sc_ref[pl.ds(i, 1), :] = jnp.reshape(w * inv_pen, (1, 1))
        lp_ref[pl.ds(i, 1), :] = jnp.reshape(w, (1, 1))
        tok_ref[pl.ds(i, 1), :] = jnp.reshape(tok, (1, 1))
        row_ref[pl.ds(i, 1), :] = jnp.reshape(r + pid * _BEAM, (1, 1))
        fin_ref[pl.ds(i, 1), :] = jnp.reshape(
            (tok == _END).astype(jnp.int32), (1, 1))
        gbrow = jnp.sum(
            jnp.where(gbrow8 == r, gb_ref[...], 0), axis=0, keepdims=True)
        nb_ref[pl.ds(i, 1), pl.ds(0, 16)] = gbrow
        nb_ref[pl.ds(i, 1), pl.ds(16, 1)] = jnp.reshape(tok, (1, 1))


def kernel(logits, topk_log_probs, growing_beam, step):
    nrows = logits.shape[0]
    nb = nrows // _BEAM
    cur_len = growing_beam.shape[1]
    step2d = jnp.reshape(jnp.asarray(step, jnp.int32), (1, 1))
    length_penalty = ((5.0 + (jnp.asarray(step, jnp.int32) + 1)) / 6.0) ** 0.95
    invpen2d = jnp.reshape(
        (1.0 / length_penalty).astype(jnp.float32), (1, 1))
    tlp2d = jnp.reshape(topk_log_probs, (nrows, 1))

    out_shape = [
        jax.ShapeDtypeStruct((nrows, 1), jnp.float32),
        jax.ShapeDtypeStruct((nrows, 1), jnp.float32),
        jax.ShapeDtypeStruct((nrows, 1), jnp.int32),
        jax.ShapeDtypeStruct((nrows, 1), jnp.int32),
        jax.ShapeDtypeStruct((nrows, cur_len + 1), jnp.int32),
        jax.ShapeDtypeStruct((nrows, 1), jnp.int32),
    ]
    sc, lp, tok, rows, nbm, fin = pl.pallas_call(
        _body,
        grid=(nb,),
        in_specs=[
            pl.BlockSpec(memory_space=pltpu.SMEM),
            pl.BlockSpec(memory_space=pltpu.SMEM),
            pl.BlockSpec((_BEAM, _VOCAB), lambda b: (b, 0)),
            pl.BlockSpec((_BEAM, 1), lambda b: (b, 0)),
            pl.BlockSpec((_BEAM, cur_len), lambda b: (b, 0)),
        ],
        out_specs=[
            pl.BlockSpec((_BEAM, 1), lambda b: (b, 0)),
            pl.BlockSpec((_BEAM, 1), lambda b: (b, 0)),
            pl.BlockSpec((_BEAM, 1), lambda b: (b, 0)),
            pl.BlockSpec((_BEAM, 1), lambda b: (b, 0)),
            pl.BlockSpec((_BEAM, cur_len + 1), lambda b: (b, 0)),
            pl.BlockSpec((_BEAM, 1), lambda b: (b, 0)),
        ],
        scratch_shapes=[
            pltpu.VMEM((2 * _BEAM, _W), jnp.float32),
            pltpu.VMEM((2 * _BEAM, _W), jnp.int32),
            pltpu.VMEM((_BEAM, _W), jnp.float32),
        ],
        out_shape=out_shape,
    )(step2d, invpen2d, logits, tlp2d, growing_beam)

    return (sc.reshape(nb, _BEAM), lp.reshape(nb, _BEAM),
            tok.reshape(nb, _BEAM), rows.reshape(-1), nbm,
            (fin.reshape(nb, _BEAM) != 0))


# speedup vs baseline: 2.8338x; 1.1358x over previous
import functools

import jax
import jax.numpy as jnp
from jax.experimental import pallas as pl
from jax.experimental.pallas import tpu as pltpu

_BEAM = 8
_VOCAB = 100000
_W = 512
_NFULL = _VOCAB // _W
_TAIL = _VOCAB - _NFULL * _W
_END = 2
_MINLEN = 10
_NEG = -1e30
_BIGI = 1 << 30


def _body(step_ref, pen_ref, x_ref, tlp_ref, gb_ref,
          sc_ref, lp_ref, tok_ref, row_ref, nb_ref, fin_ref,
          lps_ref, gs_ref, mb_ref):
    step = step_ref[0, 0]
    inv_pen = pen_ref[0, 0]
    masking = step < _MINLEN
    pid = pl.program_id(0)

    lane = jax.lax.broadcasted_iota(jnp.int32, (_BEAM, _W), 1)
    row8 = jax.lax.broadcasted_iota(jnp.int32, (_BEAM, _W), 0)

    x0 = x_ref[:, pl.ds(0, _W)]
    x0 = jnp.where((lane == _END) & masking, jnp.float32(-1e20), x0)
    v1, b1 = x0, jnp.zeros((_BEAM, _W), jnp.int32)
    v2 = jnp.full((_BEAM, _W), _NEG, jnp.float32)
    b2 = jnp.zeros((_BEAM, _W), jnp.int32)

    def upd(base, x, carry):
        v1, b1, v2, b2 = carry
        gt1 = x > v1
        gt2 = x > v2
        nv1 = jnp.maximum(x, v1)
        nb1 = jnp.where(gt1, base, b1)
        nv2 = jnp.where(gt1, v1, jnp.where(gt2, x, v2))
        nb2 = jnp.where(gt1, b1, jnp.where(gt2, base, b2))
        return nv1, nb1, nv2, nb2

    def stepA(i, carry):
        c = 1 + 2 * i
        base = pl.multiple_of(c * _W, _W)
        carry = upd(base, x_ref[:, pl.ds(base, _W)], carry)
        base2 = pl.multiple_of((c + 1) * _W, _W)
        return upd(base2, x_ref[:, pl.ds(base2, _W)], carry)

    v1, b1, v2, b2 = jax.lax.fori_loop(0, (_NFULL - 1) // 2, stepA,
                                       (v1, b1, v2, b2))

    xt = x_ref[:, pl.ds(_NFULL * _W, _TAIL)]
    xt = jnp.concatenate(
        [xt, jnp.full((_BEAM, _W - _TAIL), _NEG, jnp.float32)], axis=1)
    v1, b1, v2, b2 = upd(_NFULL * _W, xt, (v1, b1, v2, b2))

    m_row = jnp.max(v1, axis=1, keepdims=True)
    m_row = jnp.maximum(m_row, x_ref[:, pl.ds(_END, 1)])
    mb_ref[...] = jnp.broadcast_to(m_row, (_BEAM, _W))
    m_bc = mb_ref[...]

    def stepB(i, accs):
        a0, a1 = accs
        base = pl.multiple_of(2 * i * _W, _W)
        a0 = a0 + jnp.exp(x_ref[:, pl.ds(base, _W)] - m_bc)
        base2 = pl.multiple_of((2 * i + 1) * _W, _W)
        a1 = a1 + jnp.exp(x_ref[:, pl.ds(base2, _W)] - m_bc)
        return (a0, a1)

    z = jnp.zeros((_BEAM, _W), jnp.float32)
    a0, a1 = jax.lax.fori_loop(0, _NFULL // 2, stepB, (z, z))
    acc = a0 + a1
    acc = acc + jnp.exp(x_ref[:, pl.ds((_NFULL - 1) * _W, _W)] - m_bc)
    acc = acc + jnp.exp(xt - m_bc)
    logS = jnp.log(jnp.sum(acc, axis=1, keepdims=True))

    tlp = tlp_ref[...]
    lp1 = ((v1 - m_row) - logS) + tlp
    lp2 = ((v2 - m_row) - logS) + tlp
    g1 = row8 * _VOCAB + b1 + lane
    g2 = row8 * _VOCAB + b2 + lane
    lps_ref[...] = jnp.concatenate([lp1, lp2], axis=0)
    gs_ref[...] = jnp.concatenate([g1, g2], axis=0)

    lane16 = jax.lax.broadcasted_iota(jnp.int32, (2 * _BEAM, _W), 1)
    row16 = jax.lax.broadcasted_iota(jnp.int32, (2 * _BEAM, _W), 0)
    gbrow8 = jax.lax.broadcasted_iota(jnp.int32, (_BEAM, 16), 0)

    winners_b = []
    picked_g = []

    for i in range(_BEAM):
        lp_all = lps_ref[...]
        g_all = gs_ref[...]
        w = jnp.max(lp_all)
        gw = jnp.min(jnp.where(lp_all == w, g_all, _BIGI))
        r = gw // _VOCAB
        pos = gw - r * _VOCAB
        l = pos % _W
        bid = r * _W + l
        picked_g.append(gw)

        lps_ref[...] = jnp.where(g_all == gw, jnp.float32(_NEG), lp_all)

        exhausted = jnp.bool_(False)
        for pb in winners_b:
            exhausted = jnp.logical_or(exhausted, pb == bid)
        winners_b.append(bid)

        if i > 0:
            @pl.when(exhausted)
            def _rescan(r=r, l=l, picked=tuple(picked_g)):
                def excl(base, x):
                    g = row8 * _VOCAB + base + lane
                    m = masking & (base + lane == _END)
                    for pg in picked:
                        m = m | (g == pg)
                    return jnp.where(m, jnp.float32(_NEG), x)

                def stepR(c, carry):
                    vm, jm = carry
                    base = pl.multiple_of(c * _W, _W)
                    x = excl(base, x_ref[:, pl.ds(base, _W)])
                    upd_ = x > vm
                    return jnp.maximum(x, vm), jnp.where(upd_, base, jm)

                vm = jnp.full((_BEAM, _W), _NEG, jnp.float32)
                jm = jnp.zeros((_BEAM, _W), jnp.int32)
                vm, jm = jax.lax.fori_loop(0, _NFULL, stepR, (vm, jm))
                xte = excl(_NFULL * _W, xt)
                updt = xte > vm
                vm = jnp.maximum(xte, vm)
                jm = jnp.where(updt, _NFULL * _W, jm)

                lp_new = ((vm - m_row) - logS) + tlp
                g_new = row8 * _VOCAB + jm + lane
                sel = (row16 == r) & (lane16 == l)
                lps_ref[...] = jnp.where(
                    sel, jnp.tile(lp_new, (2, 1)), lps_ref[...])
                gs_ref[...] = jnp.where(
                    sel, jnp.tile(g_new, (2, 1)), gs_ref[...])

        tok = pos
        sc_ref[pl.ds(i, 1), :] = jnp.reshape(w * inv_pen, (1, 1))
        lp_ref[pl.ds(i, 1), :] = jnp.reshape(w, (1, 1))
        tok_ref[pl.ds(i, 1), :] = jnp.reshape(tok, (1, 1))
        row_ref[pl.ds(i, 1), :] = jnp.reshape(r + pid * _BEAM, (1, 1))
        fin_ref[pl.ds(i, 1), :] = jnp.reshape(
            (tok == _END).astype(jnp.int32), (1, 1))
        gbrow = jnp.sum(
            jnp.where(gbrow8 == r, gb_ref[...], 0), axis=0, keepdims=True)
        nb_ref[pl.ds(i, 1), pl.ds(0, 16)] = gbrow
        nb_ref[pl.ds(i, 1), pl.ds(16, 1)] = jnp.reshape(tok, (1, 1))


def kernel(logits, topk_log_probs, growing_beam, step):
    nrows = logits.shape[0]
    nb = nrows // _BEAM
    cur_len = growing_beam.shape[1]
    step2d = jnp.reshape(jnp.asarray(step, jnp.int32), (1, 1))
    length_penalty = ((5.0 + (jnp.asarray(step, jnp.int32) + 1)) / 6.0) ** 0.95
    invpen2d = jnp.reshape(
        (1.0 / length_penalty).astype(jnp.float32), (1, 1))
    tlp2d = jnp.reshape(topk_log_probs, (nrows, 1))

    out_shape = [
        jax.ShapeDtypeStruct((nrows, 1), jnp.float32),
        jax.ShapeDtypeStruct((nrows, 1), jnp.float32),
        jax.ShapeDtypeStruct((nrows, 1), jnp.int32),
        jax.ShapeDtypeStruct((nrows, 1), jnp.int32),
        jax.ShapeDtypeStruct((nrows, cur_len + 1), jnp.int32),
        jax.ShapeDtypeStruct((nrows, 1), jnp.int32),
    ]
    sc, lp, tok, rows, nbm, fin = pl.pallas_call(
        _body,
        grid=(nb,),
        in_specs=[
            pl.BlockSpec(memory_space=pltpu.SMEM),
            pl.BlockSpec(memory_space=pltpu.SMEM),
            pl.BlockSpec((_BEAM, _VOCAB), lambda b: (b, 0)),
            pl.BlockSpec((_BEAM, 1), lambda b: (b, 0)),
            pl.BlockSpec((_BEAM, cur_len), lambda b: (b, 0)),
        ],
        out_specs=[
            pl.BlockSpec((_BEAM, 1), lambda b: (b, 0)),
            pl.BlockSpec((_BEAM, 1), lambda b: (b, 0)),
            pl.BlockSpec((_BEAM, 1), lambda b: (b, 0)),
            pl.BlockSpec((_BEAM, 1), lambda b: (b, 0)),
            pl.BlockSpec((_BEAM, cur_len + 1), lambda b: (b, 0)),
            pl.BlockSpec((_BEAM, 1), lambda b: (b, 0)),
        ],
        scratch_shapes=[
            pltpu.VMEM((2 * _BEAM, _W), jnp.float32),
            pltpu.VMEM((2 * _BEAM, _W), jnp.int32),
            pltpu.VMEM((_BEAM, _W), jnp.float32),
        ],
        out_shape=out_shape,
        compiler_params=pltpu.CompilerParams(
            dimension_semantics=("parallel",)),
    )(step2d, invpen2d, logits, tlp2d, growing_beam)

    return (sc.reshape(nb, _BEAM), lp.reshape(nb, _BEAM),
            tok.reshape(nb, _BEAM), rows.reshape(-1), nbm,
            (fin.reshape(nb, _BEAM) != 0))
